# Initial kernel scaffold; baseline (speedup 1.0000x reference)
import jax
import jax.numpy as jnp
from jax.experimental import pallas as pl


def kernel(inputs, embeddings):
    raise NotImplementedError("write your pallas kernel here")


# speedup vs baseline: 1.1018x; 1.1018x over previous
import functools

import jax
import jax.numpy as jnp
from jax import lax
from jax.experimental import pallas as pl
from jax.experimental.pallas import tpu as pltpu
from jax.experimental.pallas import tpu_sc as plsc

_NC, _NS = 2, 16
_NW = _NC * _NS
_CHUNK = 1600


@functools.lru_cache(maxsize=None)
def _build_gather(V, D, B, chunk):
    b_per_w = B // _NW
    n_chunks = b_per_w // chunk
    mesh = plsc.VectorSubcoreMesh(core_axis_name="c", subcore_axis_name="s")

    @functools.partial(
        pl.kernel,
        mesh=mesh,
        out_type=jax.ShapeDtypeStruct((B, D), jnp.float32),
        compiler_params=pltpu.CompilerParams(use_tc_tiling_on_sc=False),
        scratch_types=[
            pltpu.VMEM((chunk,), jnp.int32),
            pltpu.VMEM((chunk, D), jnp.float32),
            pltpu.SemaphoreType.DMA,
        ],
    )
    def gather_k(table_hbm, idx_hbm, out_hbm, idx_v, rows_v, sem):
        wid = lax.axis_index("s") * _NC + lax.axis_index("c")
        base = wid * b_per_w

        @pl.loop(0, n_chunks)
        def _(c):
            off = base + c * chunk
            pltpu.sync_copy(idx_hbm.at[pl.ds(off, chunk)], idx_v)
            pltpu.async_copy(table_hbm.at[idx_v], rows_v, sem).wait()
            pltpu.sync_copy(rows_v, out_hbm.at[pl.ds(off, chunk)])

    return gather_k


def kernel(inputs, embeddings):
    batch, hist = inputs.shape
    V, D = embeddings.shape
    B = batch * hist
    idx = inputs.reshape(B).astype(jnp.int32)
    out = _build_gather(V, D, B, _CHUNK)(embeddings, idx)
    return out.reshape(batch, hist, D)


# speedup vs baseline: 1.1124x; 1.0097x over previous
import functools

import jax
import jax.numpy as jnp
from jax import lax
from jax.experimental import pallas as pl
from jax.experimental.pallas import tpu as pltpu
from jax.experimental.pallas import tpu_sc as plsc

_NC, _NS = 2, 16
_NW = _NC * _NS
_CHUNK = 1600


@functools.lru_cache(maxsize=None)
def _build_gather(V, D, B, chunk):
    b_per_w = B // _NW
    n = b_per_w // chunk
    mesh = plsc.VectorSubcoreMesh(core_axis_name="c", subcore_axis_name="s")

    @functools.partial(
        pl.kernel,
        mesh=mesh,
        out_type=jax.ShapeDtypeStruct((B, D), jnp.float32),
        compiler_params=pltpu.CompilerParams(use_tc_tiling_on_sc=False),
        scratch_types=[
            pltpu.VMEM((chunk,), jnp.int32),
            pltpu.VMEM((chunk,), jnp.int32),
            pltpu.VMEM((chunk, D), jnp.float32),
            pltpu.VMEM((chunk, D), jnp.float32),
            pltpu.SemaphoreType.DMA,
            pltpu.SemaphoreType.DMA,
            pltpu.SemaphoreType.DMA,
            pltpu.SemaphoreType.DMA,
            pltpu.SemaphoreType.DMA,
            pltpu.SemaphoreType.DMA,
        ],
    )
    def gather_k(table_hbm, idx_hbm, out_hbm,
                 idx_v0, idx_v1, rows_v0, rows_v1,
                 si0, si1, sg0, sg1, ss0, ss1):
        idx_v = (idx_v0, idx_v1)
        rows_v = (rows_v0, rows_v1)
        si = (si0, si1)
        sg = (sg0, sg1)
        ss = (ss0, ss1)

        wid = lax.axis_index("s") * _NC + lax.axis_index("c")
        base = wid * b_per_w

        def idx_load(c, b):
            return pltpu.async_copy(
                idx_hbm.at[pl.ds(base + c * chunk, chunk)], idx_v[b], si[b])

        def gather(b):
            return pltpu.async_copy(table_hbm.at[idx_v[b]], rows_v[b], sg[b])

        def store(c, b):
            return pltpu.async_copy(
                rows_v[b], out_hbm.at[pl.ds(base + c * chunk, chunk)], ss[b])

        h_i = [None] * n
        h_g = [None] * n
        h_s = [None] * n

        h_i[0] = idx_load(0, 0)
        if n > 1:
            h_i[1] = idx_load(1, 1)

        for c in range(n):
            b = c % 2
            if c >= 2:
                h_s[c - 2].wait()
            h_i[c].wait()
            h_g[c] = gather(b)
            if c >= 1:
                h_g[c - 1].wait()
                h_s[c - 1] = store(c - 1, (c - 1) % 2)
                if c + 1 < n:
                    h_i[c + 1] = idx_load(c + 1, (c + 1) % 2)

        h_g[n - 1].wait()
        h_s[n - 1] = store(n - 1, (n - 1) % 2)
        if n >= 2:
            h_s[n - 2].wait()
        h_s[n - 1].wait()

    return gather_k


def kernel(inputs, embeddings):
    batch, hist = inputs.shape
    V, D = embeddings.shape
    B = batch * hist
    idx = inputs.reshape(B).astype(jnp.int32)
    out = _build_gather(V, D, B, _CHUNK)(embeddings, idx)
    return out.reshape(batch, hist, D)
